# baseline (speedup 1.0000x reference)
import jax
import jax.numpy as jnp
from jax.experimental import pallas as pl

_FB = 256


def _blend_body(y_ref, t_ref, x_ref, o_ref):
    m = (y_ref[...] == 0).astype(jnp.float32)
    t = t_ref[...]
    xv = x_ref[...]
    o_ref[...] = xv - ((xv + 0.75) * m) * t


def kernel(x, y, template):
    B, C, S, _ = x.shape
    F = C * S * S
    xt = x.transpose(1, 2, 3, 0).reshape(F, B)
    yt = y.reshape(1, B)
    tcol = template.reshape(S * S, 1)
    nper = (S * S) // _FB
    out = pl.pallas_call(
        _blend_body,
        grid=(F // _FB,),
        in_specs=[
            pl.BlockSpec((1, B), lambda i: (0, 0)),
            pl.BlockSpec((_FB, 1), lambda i: (i % nper, 0)),
            pl.BlockSpec((_FB, B), lambda i: (i, 0)),
        ],
        out_specs=pl.BlockSpec((_FB, B), lambda i: (i, 0)),
        out_shape=jax.ShapeDtypeStruct((F, B), x.dtype),
    )(yt, tcol, xt)
    return (out.reshape(C, S, S, B).transpose(3, 0, 1, 2), y)


# speedup vs baseline: 1.0283x; 1.0283x over previous
import jax
import jax.numpy as jnp
from jax.experimental import pallas as pl

_FB = 512


def _blend_body(y_ref, t_ref, x_ref, o_ref):
    m = (y_ref[...] == 0).astype(jnp.float32)
    t = t_ref[...]
    xv = x_ref[...]
    o_ref[...] = xv - ((xv + 0.75) * m) * t


def kernel(x, y, template):
    B, C, S, _ = x.shape
    F = C * S * S
    xt = x.transpose(1, 2, 3, 0).reshape(F, B)
    yt = y.reshape(1, B)
    tcol = template.reshape(S * S, 1)
    nper = (S * S) // _FB
    out = pl.pallas_call(
        _blend_body,
        grid=(F // _FB,),
        in_specs=[
            pl.BlockSpec((1, B), lambda i: (0, 0)),
            pl.BlockSpec((_FB, 1), lambda i: (i % nper, 0)),
            pl.BlockSpec((_FB, B), lambda i: (i, 0)),
        ],
        out_specs=pl.BlockSpec((_FB, B), lambda i: (i, 0)),
        out_shape=jax.ShapeDtypeStruct((F, B), x.dtype),
    )(yt, tcol, xt)
    return (out.reshape(C, S, S, B).transpose(3, 0, 1, 2), y)


# speedup vs baseline: 1.0374x; 1.0089x over previous
import jax
import jax.numpy as jnp
from jax.experimental import pallas as pl

_FB = 512


def _blend_body(y_ref, t_ref, x_ref, o_ref):
    m = (y_ref[...] == 0).astype(jnp.float32)
    t = t_ref[:, :1]
    xv = x_ref[...]
    o_ref[...] = xv - ((xv + 0.75) * m) * t


def kernel(x, y, template):
    B, C, S, _ = x.shape
    F = C * S * S
    xt = x.transpose(1, 2, 3, 0).reshape(F, B)
    yt = y.reshape(1, B)
    tcol = jnp.broadcast_to(template.reshape(S * S, 1), (S * S, 128))
    nper = (S * S) // _FB
    out = pl.pallas_call(
        _blend_body,
        grid=(F // _FB,),
        in_specs=[
            pl.BlockSpec((1, B), lambda i: (0, 0)),
            pl.BlockSpec((_FB, 128), lambda i: (i % nper, 0)),
            pl.BlockSpec((_FB, B), lambda i: (i, 0)),
        ],
        out_specs=pl.BlockSpec((_FB, B), lambda i: (i, 0)),
        out_shape=jax.ShapeDtypeStruct((F, B), x.dtype),
    )(yt, tcol, xt)
    return (out.reshape(C, S, S, B).transpose(3, 0, 1, 2), y)
